# baseline (speedup 1.0000x reference)
import functools

import jax
import jax.numpy as jnp
from jax import lax
from jax.experimental import pallas as pl
from jax.experimental.pallas import tpu as pltpu
from jax.experimental.pallas import tpu_sc as plsc

_INFO = plsc.get_sparse_core_info()
_NC = _INFO.num_cores
_NS = _INFO.num_subcores
_NW = _NC * _NS

_CHUNK = 32


def _make_sc_copy(T, D, dtype):
    rows_per_w = T // _NW
    n_chunks = rows_per_w // _CHUNK
    mesh = plsc.VectorSubcoreMesh(core_axis_name="c", subcore_axis_name="s")

    @functools.partial(
        pl.kernel,
        mesh=mesh,
        out_type=jax.ShapeDtypeStruct((T, D), dtype),
        scratch_types=[
            pltpu.VMEM((_CHUNK, D), dtype),
            pltpu.VMEM((_CHUNK, D), dtype),
            pltpu.SemaphoreType.DMA,
            pltpu.SemaphoreType.DMA,
        ],
    )
    def sc_copy(emb_hbm, out_hbm, buf0, buf1, sem0, sem1):
        wid = lax.axis_index("s") * _NC + lax.axis_index("c")
        base = wid * rows_per_w
        bufs = (buf0, buf1)
        sems = (sem0, sem1)
        out_copies = [None] * n_chunks
        for i in range(n_chunks):
            b = i % 2
            if i >= 2:
                out_copies[i - 2].wait()
            pltpu.sync_copy(emb_hbm.at[pl.ds(base + i * _CHUNK, _CHUNK), :], bufs[b])
            out_copies[i] = pltpu.async_copy(
                bufs[b], out_hbm.at[pl.ds(base + i * _CHUNK, _CHUNK), :], sems[b]
            )
        for i in range(max(0, n_chunks - 2), n_chunks):
            out_copies[i].wait()

    return sc_copy


def kernel(x, emb):
    T = x.shape[1]
    D = emb.shape[1]
    assert T % (_NW * _CHUNK) == 0
    out = _make_sc_copy(T, D, emb.dtype)(emb[:T])
    return out[None, :, :]


# speedup vs baseline: 1.0326x; 1.0326x over previous
import functools

import jax
import jax.numpy as jnp
from jax import lax
from jax.experimental import pallas as pl
from jax.experimental.pallas import tpu as pltpu
from jax.experimental.pallas import tpu_sc as plsc

_INFO = plsc.get_sparse_core_info()
_NC = _INFO.num_cores
_NS = _INFO.num_subcores
_NW = _NC * _NS

_CHUNK = 64


def _make_sc_copy(T, D, dtype):
    rows_per_w = T // _NW
    n_chunks = rows_per_w // _CHUNK
    mesh = plsc.VectorSubcoreMesh(core_axis_name="c", subcore_axis_name="s")

    @functools.partial(
        pl.kernel,
        mesh=mesh,
        out_type=jax.ShapeDtypeStruct((T, D), dtype),
        scratch_types=[
            pltpu.VMEM((_CHUNK, D), dtype),
            pltpu.VMEM((_CHUNK, D), dtype),
            pltpu.SemaphoreType.DMA,
            pltpu.SemaphoreType.DMA,
        ],
    )
    def sc_copy(emb_hbm, out_hbm, buf0, buf1, sem0, sem1):
        wid = lax.axis_index("s") * _NC + lax.axis_index("c")
        base = wid * rows_per_w
        bufs = (buf0, buf1)
        sems = (sem0, sem1)
        out_copies = [None] * n_chunks
        for i in range(n_chunks):
            b = i % 2
            if i >= 2:
                out_copies[i - 2].wait()
            pltpu.sync_copy(emb_hbm.at[pl.ds(base + i * _CHUNK, _CHUNK), :], bufs[b])
            out_copies[i] = pltpu.async_copy(
                bufs[b], out_hbm.at[pl.ds(base + i * _CHUNK, _CHUNK), :], sems[b]
            )
        for i in range(max(0, n_chunks - 2), n_chunks):
            out_copies[i].wait()

    return sc_copy


def kernel(x, emb):
    T = x.shape[1]
    D = emb.shape[1]
    assert T % (_NW * _CHUNK) == 0
    out = _make_sc_copy(T, D, emb.dtype)(emb[:T])
    return out[None, :, :]


# speedup vs baseline: 2.5019x; 2.4230x over previous
import jax
import jax.numpy as jnp
from jax.experimental import pallas as pl
from jax.experimental.pallas import tpu as pltpu

_BLOCK = 4096


def _copy_block(emb_ref, out_ref):
    out_ref[0, :, :] = emb_ref[:, :]


def kernel(x, emb):
    T = x.shape[1]
    D = emb.shape[1]
    assert T % _BLOCK == 0
    out = pl.pallas_call(
        _copy_block,
        grid=(T // _BLOCK,),
        in_specs=[pl.BlockSpec((_BLOCK, D), lambda i: (i, 0))],
        out_specs=pl.BlockSpec((1, _BLOCK, D), lambda i: (0, i, 0)),
        out_shape=jax.ShapeDtypeStruct((1, T, D), emb.dtype),
        compiler_params=pltpu.CompilerParams(
            dimension_semantics=("parallel",),
        ),
    )(emb[:T])
    return out


# speedup vs baseline: 2.5127x; 1.0043x over previous
import jax
import jax.numpy as jnp
from jax.experimental import pallas as pl
from jax.experimental.pallas import tpu as pltpu

_BLOCK = 4096


def _copy_block(emb_ref, out_ref):
    out_ref[:, :] = emb_ref[:, :]


def kernel(x, emb):
    T = x.shape[1]
    D = emb.shape[1]
    assert T % _BLOCK == 0
    out = pl.pallas_call(
        _copy_block,
        grid=(T // _BLOCK,),
        in_specs=[pl.BlockSpec((_BLOCK, D), lambda i: (i, 0))],
        out_specs=pl.BlockSpec((_BLOCK, D), lambda i: (i, 0)),
        out_shape=jax.ShapeDtypeStruct((T, D), emb.dtype),
        compiler_params=pltpu.CompilerParams(
            dimension_semantics=("parallel",),
        ),
    )(emb[:T])
    return out[None, :, :]
